# baseline (speedup 1.0000x reference)
import functools

import jax
import jax.numpy as jnp
from jax import lax
from jax.experimental import pallas as pl
from jax.experimental.pallas import tpu as pltpu
from jax.experimental.pallas import tpu_sc as plsc

N = 10000
E = 320000
D = 128
T = 2
S = 512
K = 64

NC = 2
NS = 16
L = 16

EC = 192
EPT_CH = 54
EPT = EPT_CH * EC
E_PAD = EPT * NS * NC
SGT = S // NS
NPT = N // NS
ZR = NPT // EC
ZREM = NPT - ZR * EC


def _sc_body(xbar_hbm, src_hbm, dst_hbm, w_hbm, subg_hbm, out_hbm,
             agg_sp,
             gbufa, gbufb, srca, srcb, wa, wb, dst0, dst1, dst2,
             gsem0, gsem1, ssem0, ssem1):
    cid = lax.axis_index("c")
    sid = lax.axis_index("s")
    gbuf = (gbufa, gbufb)
    srwar = (srca, srcb)
    war = (wa, wb)
    dstr = (dst0, dst1, dst2)
    gsem = (gsem0, gsem1)
    ssem = (ssem0, ssem1)

    def _zero_row(i, _):
        for u in range(D // L):
            gbufa[i, u * L:(u + 1) * L] = jnp.zeros((L,), jnp.float32)
        return 0

    lax.fori_loop(0, EC, _zero_row, 0)
    r0 = sid * NPT
    for z in range(ZR):
        pltpu.sync_copy(gbufa, agg_sp.at[pl.ds(r0 + z * EC, EC), :])
    pltpu.sync_copy(gbufa.at[pl.ds(0, ZREM), :],
                    agg_sp.at[pl.ds(r0 + ZR * EC, ZREM), :])

    plsc.subcore_barrier()

    ebase = cid * (NS * EPT) + sid * EPT

    pltpu.sync_copy(src_hbm.at[pl.ds(ebase, EC)], srca)
    pltpu.sync_copy(w_hbm.at[pl.ds(ebase, EC)], wa)
    pltpu.sync_copy(dst_hbm.at[pl.ds(ebase, EC)], dst0)
    pltpu.async_copy(xbar_hbm.at[srca], gbufa, gsem0)

    def _six(i, _):
        for j in range(6):
            g = i * 6 + j
            bx = j % 2
            by = 1 - bx
            q1 = (j + 1) % 3

            @pl.when(g + 1 < EPT_CH)
            def _():
                nxt = ebase + (g + 1) * EC
                pltpu.sync_copy(src_hbm.at[pl.ds(nxt, EC)], srwar[by])
                pltpu.sync_copy(w_hbm.at[pl.ds(nxt, EC)], war[by])
                pltpu.sync_copy(dst_hbm.at[pl.ds(nxt, EC)], dstr[q1])

            @pl.when(g >= 1)
            def _():
                pltpu.make_async_copy(gbuf[by], agg_sp.at[dstr[(j + 2) % 3]],
                                      ssem[by]).wait()

            @pl.when(g + 1 < EPT_CH)
            def _():
                pltpu.async_copy(xbar_hbm.at[srwar[by]], gbuf[by], gsem[by])

            pltpu.make_async_copy(xbar_hbm.at[srwar[bx]], gbuf[bx],
                                  gsem[bx]).wait()

            def _scale(grp, _):
                wvec = war[bx][pl.ds(grp * L, L)]
                for el in range(L):
                    e = grp * L + el
                    ws = wvec[el]
                    for u in range(D // L):
                        s = pl.ds(u * L, L)
                        gbuf[bx][e, s] = gbuf[bx][e, s] * ws
                return 0

            lax.fori_loop(0, EC // L, _scale, 0)

            pltpu.async_copy(gbuf[bx], agg_sp.at[dstr[j % 3]], ssem[bx],
                             add=True)
        return 0

    lax.fori_loop(0, EPT_CH // 6, _six, 0)

    pltpu.make_async_copy(gbufb, agg_sp.at[dst2], ssem1).wait()

    plsc.subcore_barrier()

    sgbase = sid * SGT
    pltpu.sync_copy(subg_hbm.at[pl.ds(sgbase * K, K)],
                    srca.at[pl.ds(0, K)])
    pltpu.async_copy(agg_sp.at[srca.at[pl.ds(0, K)]],
                     gbufa.at[pl.ds(0, K), :], gsem0)

    def _pool2(i, _):
        for j in range(2):
            q = i * 2 + j
            bx = j
            by = 1 - j

            @pl.when(q + 1 < SGT)
            def _():
                pltpu.sync_copy(
                    subg_hbm.at[pl.ds((sgbase + q + 1) * K, K)],
                    srwar[by].at[pl.ds(0, K)])
                pltpu.async_copy(agg_sp.at[srwar[by].at[pl.ds(0, K)]],
                                 gbuf[by].at[pl.ds(0, K), :], gsem[by])

            pltpu.make_async_copy(agg_sp.at[srwar[bx].at[pl.ds(0, K)]],
                                  gbuf[bx].at[pl.ds(0, K), :],
                                  gsem[bx]).wait()

            def _acc(r, carry):
                return tuple(
                    carry[u] + gbuf[bx][r, u * L:(u + 1) * L]
                    for u in range(D // L))

            acc = lax.fori_loop(
                0, K, _acc, tuple(jnp.zeros((L,), jnp.float32)
                                  for _ in range(D // L)))
            for u in range(D // L):
                gbufa[K + q, u * L:(u + 1) * L] = acc[u]
        return 0

    lax.fori_loop(0, SGT // 2, _pool2, 0)
    pltpu.sync_copy(gbufa.at[pl.ds(K, SGT), :],
                    out_hbm.at[cid, pl.ds(sid * SGT, SGT), :])


_sc_call = functools.partial(
    pl.kernel,
    out_type=jax.ShapeDtypeStruct((NC, S, D), jnp.float32),
    mesh=plsc.VectorSubcoreMesh(core_axis_name="c", subcore_axis_name="s"),
    scratch_types=[
        pltpu.VMEM_SHARED((N, D), jnp.float32),
        pltpu.VMEM((EC, D), jnp.float32),
        pltpu.VMEM((EC, D), jnp.float32),
        pltpu.VMEM((EC,), jnp.int32),
        pltpu.VMEM((EC,), jnp.int32),
        pltpu.VMEM((EC,), jnp.float32),
        pltpu.VMEM((EC,), jnp.float32),
        pltpu.VMEM((EC,), jnp.int32),
        pltpu.VMEM((EC,), jnp.int32),
        pltpu.VMEM((EC,), jnp.int32),
        pltpu.SemaphoreType.DMA,
        pltpu.SemaphoreType.DMA,
        pltpu.SemaphoreType.DMA,
        pltpu.SemaphoreType.DMA,
    ],
)(_sc_body)


MBLK = 1000


def _mean_body(x_ref, o_ref):
    o_ref[...] = (x_ref[:, 0, :] + x_ref[:, 1, :]) * 0.5


_mean_call = pl.pallas_call(
    _mean_body,
    grid=(N // MBLK,),
    in_specs=[pl.BlockSpec((MBLK, T, D), lambda i: (i, 0, 0))],
    out_specs=pl.BlockSpec((MBLK, D), lambda i: (i, 0)),
    out_shape=jax.ShapeDtypeStruct((N, D), jnp.float32),
)


def _mm_body(pre_ref, w_ref, b_ref, o_ref):
    o_ref[...] = (jnp.dot(pre_ref[0] + pre_ref[1], w_ref[...],
                          preferred_element_type=jnp.float32)
                  + jnp.float32(K) * b_ref[...])


_mm_call = pl.pallas_call(
    _mm_body,
    out_shape=jax.ShapeDtypeStruct((S, D), jnp.float32),
)


def kernel(x, edge_index, edge_weight, subG_node, W, b):
    xbar = _mean_call(x)
    src = edge_index[0].astype(jnp.int32)
    dst = edge_index[1].astype(jnp.int32)
    w = edge_weight.astype(jnp.float32)
    pad = E_PAD - E
    src = jnp.concatenate([src, jnp.zeros((pad,), jnp.int32)])
    dst = jnp.concatenate([dst, jnp.zeros((pad,), jnp.int32)])
    w = jnp.concatenate([w, jnp.zeros((pad,), jnp.float32)])
    subg = subG_node.astype(jnp.int32).reshape(S * K)
    pre = _sc_call(xbar, src, dst, w, subg)
    return _mm_call(pre, W, b.reshape(1, D))


# speedup vs baseline: 1.3164x; 1.3164x over previous
import functools

import jax
import jax.numpy as jnp
from jax import lax
from jax.experimental import pallas as pl
from jax.experimental.pallas import tpu as pltpu
from jax.experimental.pallas import tpu_sc as plsc

N = 10000
E = 320000
D = 128
T = 2
S = 512
K = 64

NC = 2
NS = 16
L = 16

EC = 352
EPT_CH = 29
EPT = EPT_CH * EC
E_PAD = EPT * NS * NC
SGT = S // NS
NPT = N // NS
ZREM = NPT - EC


def _sc_body(xbar_hbm, ed_hbm, w_hbm, subg_hbm, out_hbm,
             agg_sp, gbuf, sdw_v, w_v, sgidx_v, sem):
    cid = lax.axis_index("c")
    sid = lax.axis_index("s")

    def _zero_row(i, _):
        for u in range(D // L):
            gbuf[i, u * L:(u + 1) * L] = jnp.zeros((L,), jnp.float32)
        return 0

    lax.fori_loop(0, EC, _zero_row, 0)
    r0 = sid * NPT
    pltpu.sync_copy(gbuf, agg_sp.at[pl.ds(r0, EC), :])
    pltpu.sync_copy(gbuf.at[pl.ds(0, ZREM), :],
                    agg_sp.at[pl.ds(r0 + EC, ZREM), :])

    plsc.subcore_barrier()

    cbase = (cid * NS + sid) * EPT_CH

    def _chunk(g, _):
        pltpu.sync_copy(ed_hbm.at[pl.ds((cbase + g) * (2 * EC), 2 * EC)],
                        sdw_v)
        pltpu.sync_copy(w_hbm.at[pl.ds((cbase + g) * EC, EC)], w_v)
        pltpu.async_copy(xbar_hbm.at[sdw_v.at[pl.ds(0, EC)]], gbuf,
                         sem).wait()

        def _scale(grp, _):
            wvec = w_v[pl.ds(grp * L, L)]
            for el in range(L):
                e = grp * L + el
                ws = wvec[el]
                for u in range(D // L):
                    s = pl.ds(u * L, L)
                    gbuf[e, s] = gbuf[e, s] * ws
            return 0

        lax.fori_loop(0, EC // L, _scale, 0)
        pltpu.sync_copy(gbuf, agg_sp.at[sdw_v.at[pl.ds(EC, EC)]], add=True)
        return 0

    lax.fori_loop(0, EPT_CH, _chunk, 0)

    plsc.subcore_barrier()

    sgbase = sid * SGT
    pltpu.sync_copy(subg_hbm.at[pl.ds(sgbase * K, SGT * K)], sgidx_v)

    def _pool(q, _):
        pltpu.async_copy(agg_sp.at[sgidx_v.at[pl.ds(q * K, K)]],
                         gbuf.at[pl.ds(0, K), :], sem).wait()

        def _acc(r, carry):
            return tuple(
                carry[u] + gbuf[r, u * L:(u + 1) * L]
                for u in range(D // L))

        acc = lax.fori_loop(
            0, K, _acc, tuple(jnp.zeros((L,), jnp.float32)
                              for _ in range(D // L)))
        for u in range(D // L):
            gbuf[K + q, u * L:(u + 1) * L] = acc[u]
        return 0

    lax.fori_loop(0, SGT, _pool, 0)
    pltpu.sync_copy(gbuf.at[pl.ds(K, SGT), :],
                    out_hbm.at[cid, pl.ds(sid * SGT, SGT), :])


_sc_call = functools.partial(
    pl.kernel,
    out_type=jax.ShapeDtypeStruct((NC, S, D), jnp.float32),
    mesh=plsc.VectorSubcoreMesh(core_axis_name="c", subcore_axis_name="s"),
    scratch_types=[
        pltpu.VMEM_SHARED((N, D), jnp.float32),
        pltpu.VMEM((EC, D), jnp.float32),
        pltpu.VMEM((2 * EC,), jnp.int32),
        pltpu.VMEM((EC,), jnp.float32),
        pltpu.VMEM((SGT * K,), jnp.int32),
        pltpu.SemaphoreType.DMA,
    ],
)(_sc_body)


MBLK = 1000


def _mean_body(x_ref, o_ref):
    o_ref[...] = (x_ref[:, 0, :] + x_ref[:, 1, :]) * 0.5


_mean_call = pl.pallas_call(
    _mean_body,
    grid=(N // MBLK,),
    in_specs=[pl.BlockSpec((MBLK, T, D), lambda i: (i, 0, 0))],
    out_specs=pl.BlockSpec((MBLK, D), lambda i: (i, 0)),
    out_shape=jax.ShapeDtypeStruct((N, D), jnp.float32),
)


def _mm_body(pre_ref, w_ref, b_ref, o_ref):
    o_ref[...] = (jnp.dot(pre_ref[0] + pre_ref[1], w_ref[...],
                          preferred_element_type=jnp.float32)
                  + jnp.float32(K) * b_ref[...])


_mm_call = pl.pallas_call(
    _mm_body,
    out_shape=jax.ShapeDtypeStruct((S, D), jnp.float32),
)


def kernel(x, edge_index, edge_weight, subG_node, W, b):
    xbar = _mean_call(x)
    src = edge_index[0].astype(jnp.int32)
    dst = edge_index[1].astype(jnp.int32)
    w = edge_weight.astype(jnp.float32)
    pad = E_PAD - E
    src = jnp.concatenate([src, jnp.zeros((pad,), jnp.int32)])
    dst = jnp.concatenate([dst, jnp.zeros((pad,), jnp.int32)])
    w = jnp.concatenate([w, jnp.zeros((pad,), jnp.float32)])
    ed = jnp.stack([src.reshape(-1, EC), dst.reshape(-1, EC)],
                   axis=1).reshape(-1)
    subg = subG_node.astype(jnp.int32).reshape(S * K)
    pre = _sc_call(xbar, ed, w, subg)
    return _mm_call(pre, W, b.reshape(1, D))


# speedup vs baseline: 1.3206x; 1.0031x over previous
import functools

import jax
import jax.numpy as jnp
from jax import lax
from jax.experimental import pallas as pl
from jax.experimental.pallas import tpu as pltpu
from jax.experimental.pallas import tpu_sc as plsc

N = 10000
E = 320000
D = 128
T = 2
S = 512
K = 64

NC = 2
NS = 16
L = 16

EC = 352
EPT_CH = 29
EPT = EPT_CH * EC
E_PAD = EPT * NS * NC
SGT = S // NS
NPT = N // NS
ZREM = NPT - EC


def _sc_body(xbar_hbm, ed_hbm, w_hbm, subg_hbm, out_hbm,
             agg_sp, gbuf, sdw_v, w_v, sgidx_v, sem):
    cid = lax.axis_index("c")
    sid = lax.axis_index("s")

    def _zero_row(i, _):
        for u in range(D // L):
            gbuf[i, u * L:(u + 1) * L] = jnp.zeros((L,), jnp.float32)
        return 0

    lax.fori_loop(0, EC, _zero_row, 0)
    r0 = sid * NPT
    pltpu.sync_copy(gbuf, agg_sp.at[pl.ds(r0, EC), :])
    pltpu.sync_copy(gbuf.at[pl.ds(0, ZREM), :],
                    agg_sp.at[pl.ds(r0 + EC, ZREM), :])

    plsc.subcore_barrier()

    cbase = (cid * NS + sid) * EPT_CH

    def _chunk(g, _):
        pltpu.sync_copy(ed_hbm.at[pl.ds((cbase + g) * (2 * EC), 2 * EC)],
                        sdw_v)
        pltpu.sync_copy(w_hbm.at[pl.ds((cbase + g) * EC, EC)], w_v)
        pltpu.async_copy(xbar_hbm.at[sdw_v.at[pl.ds(0, EC)]], gbuf,
                         sem).wait()

        def _scale(grp, _):
            wvec = w_v[pl.ds(grp * L, L)]
            for el in range(L):
                e = grp * L + el
                ws = wvec[el]
                for u in range(D // L):
                    s = pl.ds(u * L, L)
                    gbuf[e, s] = gbuf[e, s] * ws
            return 0

        lax.fori_loop(0, EC // L, _scale, 0)
        pltpu.sync_copy(gbuf, agg_sp.at[sdw_v.at[pl.ds(EC, EC)]], add=True)
        return 0

    lax.fori_loop(0, EPT_CH, _chunk, 0)

    plsc.subcore_barrier()

    sgbase = sid * SGT
    pltpu.sync_copy(subg_hbm.at[pl.ds(sgbase * K, SGT * K)], sgidx_v)

    def _pool(i, _):
        pltpu.async_copy(agg_sp.at[sgidx_v.at[pl.ds(i * (4 * K), 4 * K)]],
                         gbuf.at[pl.ds(0, 4 * K), :], sem).wait()
        for q2 in range(4):
            def _acc(r, carry):
                return tuple(
                    carry[u] + gbuf[q2 * K + r, u * L:(u + 1) * L]
                    for u in range(D // L))

            acc = lax.fori_loop(
                0, K, _acc, tuple(jnp.zeros((L,), jnp.float32)
                                  for _ in range(D // L)))
            for u in range(D // L):
                gbuf[4 * K + i * 4 + q2, u * L:(u + 1) * L] = acc[u]
        return 0

    lax.fori_loop(0, SGT // 4, _pool, 0)
    pltpu.sync_copy(gbuf.at[pl.ds(4 * K, SGT), :],
                    out_hbm.at[cid, pl.ds(sid * SGT, SGT), :])


_sc_call = functools.partial(
    pl.kernel,
    out_type=jax.ShapeDtypeStruct((NC, S, D), jnp.float32),
    mesh=plsc.VectorSubcoreMesh(core_axis_name="c", subcore_axis_name="s"),
    scratch_types=[
        pltpu.VMEM_SHARED((N, D), jnp.float32),
        pltpu.VMEM((EC, D), jnp.float32),
        pltpu.VMEM((2 * EC,), jnp.int32),
        pltpu.VMEM((EC,), jnp.float32),
        pltpu.VMEM((SGT * K,), jnp.int32),
        pltpu.SemaphoreType.DMA,
    ],
)(_sc_body)


MBLK = 1000


def _mean_body(x_ref, o_ref):
    o_ref[...] = (x_ref[:, 0, :] + x_ref[:, 1, :]) * 0.5


_mean_call = pl.pallas_call(
    _mean_body,
    grid=(N // MBLK,),
    in_specs=[pl.BlockSpec((MBLK, T, D), lambda i: (i, 0, 0))],
    out_specs=pl.BlockSpec((MBLK, D), lambda i: (i, 0)),
    out_shape=jax.ShapeDtypeStruct((N, D), jnp.float32),
)


def _mm_body(pre_ref, w_ref, b_ref, o_ref):
    o_ref[...] = (jnp.dot(pre_ref[0] + pre_ref[1], w_ref[...],
                          preferred_element_type=jnp.float32)
                  + jnp.float32(K) * b_ref[...])


_mm_call = pl.pallas_call(
    _mm_body,
    out_shape=jax.ShapeDtypeStruct((S, D), jnp.float32),
)


def kernel(x, edge_index, edge_weight, subG_node, W, b):
    xbar = _mean_call(x)
    src = edge_index[0].astype(jnp.int32)
    dst = edge_index[1].astype(jnp.int32)
    w = edge_weight.astype(jnp.float32)
    pad = E_PAD - E
    src = jnp.concatenate([src, jnp.zeros((pad,), jnp.int32)])
    dst = jnp.concatenate([dst, jnp.zeros((pad,), jnp.int32)])
    w = jnp.concatenate([w, jnp.zeros((pad,), jnp.float32)])
    ed = jnp.stack([src.reshape(-1, EC), dst.reshape(-1, EC)],
                   axis=1).reshape(-1)
    subg = subG_node.astype(jnp.int32).reshape(S * K)
    pre = _sc_call(xbar, ed, w, subg)
    return _mm_call(pre, W, b.reshape(1, D))
